# baseline (speedup 1.0000x reference)
import functools

import jax
import jax.numpy as jnp
from jax import lax
from jax.experimental import pallas as pl
from jax.experimental.pallas import tpu as pltpu
from jax.experimental.pallas import tpu_sc as plsc

D_MODEL = 768
N_TOK = 32768
NW = 32
TPW = N_TOK // NW
CH = 128
NCH = TPW // CH


def _sc_embed(seg_flat, table):
    mesh = plsc.VectorSubcoreMesh(core_axis_name="c", subcore_axis_name="s")

    @functools.partial(
        pl.kernel,
        mesh=mesh,
        out_type=jax.ShapeDtypeStruct((N_TOK, D_MODEL), jnp.float32),
        scratch_types=[
            pltpu.VMEM((TPW,), jnp.int32),
            pltpu.VMEM((CH, D_MODEL), jnp.float32),
            pltpu.SemaphoreType.DMA,
        ],
    )
    def k(seg_hbm, tab_hbm, out_hbm, idx_v, rows_v, sem):
        wid = lax.axis_index("s") * 2 + lax.axis_index("c")
        base = wid * TPW
        pltpu.sync_copy(seg_hbm.at[pl.ds(base, TPW)], idx_v)

        def body(i, carry):
            off = i * CH
            pltpu.async_copy(tab_hbm.at[idx_v.at[pl.ds(off, CH)]], rows_v, sem).wait()
            pltpu.sync_copy(rows_v, out_hbm.at[pl.ds(base + off, CH)])
            return carry

        lax.fori_loop(0, NCH, body, 0)

    return k(seg_flat, table)


def kernel(segment_ids, table):
    b, s = segment_ids.shape
    seg_flat = segment_ids.reshape(b * s).astype(jnp.int32)
    out = _sc_embed(seg_flat, table)
    return out.reshape(b, s, D_MODEL)


# speedup vs baseline: 2.6133x; 2.6133x over previous
import functools

import jax
import jax.numpy as jnp
from jax import lax
from jax.experimental import pallas as pl
from jax.experimental.pallas import tpu as pltpu
from jax.experimental.pallas import tpu_sc as plsc

D_MODEL = 768
LANES = 16
KREG = D_MODEL // LANES
N_TOK = 32768
NW = 32
TPW = N_TOK // NW
CH = 64
NCH = TPW // CH


def _sc_embed(seg_2d, table_flat):
    mesh = plsc.VectorSubcoreMesh(core_axis_name="c", subcore_axis_name="s")

    @functools.partial(
        pl.kernel,
        mesh=mesh,
        out_type=jax.ShapeDtypeStruct((N_TOK, D_MODEL), jnp.float32),
        scratch_types=[
            pltpu.VMEM((TPW,), jnp.int32),
            pltpu.VMEM((2 * D_MODEL,), jnp.float32),
            pltpu.VMEM((D_MODEL,), jnp.float32),
            pltpu.VMEM((CH, D_MODEL), jnp.float32),
            pltpu.VMEM((CH, D_MODEL), jnp.float32),
            pltpu.SemaphoreType.DMA,
            pltpu.SemaphoreType.DMA,
        ],
    )
    def k(seg_hbm, tab_hbm, out_hbm, seg_v, tab_v, dif_v, ob0, ob1, sem0, sem1):
        wid = lax.axis_index("s") * 2 + lax.axis_index("c")
        base = wid * TPW
        pltpu.sync_copy(tab_hbm, tab_v)
        pltpu.sync_copy(seg_hbm.at[wid], seg_v)
        for kk in range(KREG):
            dif_v[pl.ds(kk * LANES, LANES)] = (
                tab_v[pl.ds(D_MODEL + kk * LANES, LANES)]
                - tab_v[pl.ds(kk * LANES, LANES)]
            )

        def fill(c, ob):
            def grp(g, carry):
                fgrp = seg_v[pl.ds(c * CH + g * LANES, LANES)].astype(
                    jnp.float32
                )

                def tok(j, carry2):
                    jv = lax.broadcast(j, (LANES,))
                    fj = lax.gather(
                        fgrp,
                        jv[:, None],
                        dimension_numbers=lax.GatherDimensionNumbers(
                            offset_dims=(),
                            collapsed_slice_dims=(0,),
                            start_index_map=(0,),
                        ),
                        slice_sizes=(1,),
                        mode=lax.GatherScatterMode.PROMISE_IN_BOUNDS,
                    )
                    row = g * LANES + j
                    for kk in range(KREG):
                        t0k = tab_v[pl.ds(kk * LANES, LANES)]
                        dk = dif_v[pl.ds(kk * LANES, LANES)]
                        ob[row, pl.ds(kk * LANES, LANES)] = t0k + fj * dk
                    return carry2

                lax.fori_loop(0, LANES, tok, 0)
                return carry

            lax.fori_loop(0, CH // LANES, grp, 0)

        def flush(c, ob, sem):
            return pltpu.async_copy(
                ob, out_hbm.at[pl.ds(base + c * CH, CH)], sem
            )

        fill(0, ob0)
        h0 = flush(0, ob0, sem0)
        fill(1, ob1)
        h1 = flush(1, ob1, sem1)
        for c in range(2, NCH):
            if c % 2 == 0:
                h0.wait()
                fill(c, ob0)
                h0 = flush(c, ob0, sem0)
            else:
                h1.wait()
                fill(c, ob1)
                h1 = flush(c, ob1, sem1)
        h0.wait()
        h1.wait()

    return k(seg_2d, table_flat)


def kernel(segment_ids, table):
    b, s = segment_ids.shape
    seg_2d = segment_ids.reshape(NW, TPW).astype(jnp.int32)
    out = _sc_embed(seg_2d, table.reshape(2 * D_MODEL))
    return out.reshape(b, s, D_MODEL)
